# baseline (speedup 1.0000x reference)
import jax
import jax.numpy as jnp
from jax.experimental import pallas as pl
from jax.experimental.pallas import tpu as pltpu

_EPS = 1e-06
_ALPHA = 0.1


def _sums_body(x_ref, s_ref, s2_ref):
    x = x_ref[0]
    s_ref[0, :, 0] = jnp.sum(x, axis=1)
    s2_ref[0, :, 0] = jnp.sum(x * x, axis=1)


def _stats_body(cm_ref, lm_ref, s_ref, s2_ref, scale_ref, bias_ref):
    cm = cm_ref[0]
    s = s_ref[:, :, 0]
    s2 = s2_ref[:, :, 0]
    lm = lm_ref[:, :, 0]
    B, K = cm.shape

    ids = jnp.argmax(cm, axis=1)
    onehot = (ids[:, None] == jax.lax.broadcasted_iota(jnp.int32, (B, K), 1)).astype(jnp.float32)

    n_sp = jnp.float32(4096.0)
    sample_mu = s / n_sp
    sample_var = (s2 - n_sp * sample_mu * sample_mu) / (n_sp - 1.0)
    sample_std = jnp.sqrt(sample_var + _EPS)

    counts = jnp.sum(onehot, axis=0)
    c_sum = jax.lax.dot_general(onehot, s, (((0,), (0,)), ((), ())),
                                preferred_element_type=jnp.float32)
    c_sum2 = jax.lax.dot_general(onehot, s2, (((0,), (0,)), ((), ())),
                                 preferred_element_type=jnp.float32)
    n_c = counts * n_sp
    n_c_safe = jnp.maximum(n_c, 1.0)[:, None]
    denom = jnp.maximum(n_c - 1.0, 1.0)[:, None]
    cluster_mu = c_sum / n_c_safe
    cluster_var = (c_sum2 - n_c[:, None] * cluster_mu * cluster_mu) / denom
    cluster_std = jnp.sqrt(jnp.maximum(cluster_var, 0.0) + _EPS)

    cmu = jnp.dot(onehot, cluster_mu, preferred_element_type=jnp.float32)
    cstd = jnp.dot(onehot, cluster_std, preferred_element_type=jnp.float32)

    mu_mix = sample_mu * lm + cmu * (1.0 - lm)
    std_mix = sample_std * lm + cstd * (1.0 - lm)
    scale = std_mix / sample_std
    bias = mu_mix - sample_mu * scale
    scale_ref[:, :, 0] = scale
    bias_ref[:, :, 0] = bias


def _apply_body(x_ref, scale_ref, bias_ref, o_ref):
    o_ref[...] = x_ref[...] * scale_ref[:, :, 0:1] + bias_ref[:, :, 0:1]


def kernel(x, cluster_map):
    B, C, H, W = x.shape
    K = cluster_map.shape[2]
    HW = H * W
    xf = x.reshape(B, C, HW)

    lmda = jax.random.beta(jax.random.key(42), _ALPHA, _ALPHA, (B, 1, 1, 1)).astype(x.dtype)
    lm = lmda.reshape(B, 1, 1)

    sums, sums2 = pl.pallas_call(
        _sums_body,
        grid=(B,),
        in_specs=[pl.BlockSpec((1, C, HW), lambda i: (i, 0, 0))],
        out_specs=[pl.BlockSpec((1, C, 1), lambda i: (i, 0, 0)),
                   pl.BlockSpec((1, C, 1), lambda i: (i, 0, 0))],
        out_shape=[jax.ShapeDtypeStruct((B, C, 1), jnp.float32),
                   jax.ShapeDtypeStruct((B, C, 1), jnp.float32)],
    )(xf)

    scale, bias = pl.pallas_call(
        _stats_body,
        out_shape=[jax.ShapeDtypeStruct((B, C, 1), jnp.float32),
                   jax.ShapeDtypeStruct((B, C, 1), jnp.float32)],
    )(cluster_map, lm, sums, sums2)

    out = pl.pallas_call(
        _apply_body,
        grid=(B,),
        in_specs=[pl.BlockSpec((1, C, HW), lambda i: (i, 0, 0)),
                  pl.BlockSpec((1, C, 1), lambda i: (i, 0, 0)),
                  pl.BlockSpec((1, C, 1), lambda i: (i, 0, 0))],
        out_specs=pl.BlockSpec((1, C, HW), lambda i: (i, 0, 0)),
        out_shape=jax.ShapeDtypeStruct((B, C, HW), x.dtype),
    )(xf, scale, bias)

    return out.reshape(B, C, H, W)


# speedup vs baseline: 1.0766x; 1.0766x over previous
import jax
import jax.numpy as jnp
from jax.experimental import pallas as pl
from jax.experimental.pallas import tpu as pltpu

_EPS = 1e-06
_ALPHA = 0.1


def _sums_body(x_ref, s_ref, s2_ref):
    x = x_ref[...]
    s_ref[...] = jnp.sum(x, axis=2, keepdims=True)
    s2_ref[...] = jnp.sum(x * x, axis=2, keepdims=True)


def _stats_body(cm_ref, lm_ref, s_ref, s2_ref, scale_ref, bias_ref):
    cm = cm_ref[0]
    s = s_ref[:, :, 0]
    s2 = s2_ref[:, :, 0]
    lm = lm_ref[:, :, 0]
    B, K = cm.shape

    ids = jnp.argmax(cm, axis=1)
    onehot = (ids[:, None] == jax.lax.broadcasted_iota(jnp.int32, (B, K), 1)).astype(jnp.float32)

    n_sp = jnp.float32(4096.0)
    sample_mu = s / n_sp
    sample_var = (s2 - n_sp * sample_mu * sample_mu) / (n_sp - 1.0)
    sample_std = jnp.sqrt(sample_var + _EPS)

    counts = jnp.sum(onehot, axis=0)
    c_sum = jax.lax.dot_general(onehot, s, (((0,), (0,)), ((), ())),
                                preferred_element_type=jnp.float32)
    c_sum2 = jax.lax.dot_general(onehot, s2, (((0,), (0,)), ((), ())),
                                 preferred_element_type=jnp.float32)
    n_c = counts * n_sp
    n_c_safe = jnp.maximum(n_c, 1.0)[:, None]
    denom = jnp.maximum(n_c - 1.0, 1.0)[:, None]
    cluster_mu = c_sum / n_c_safe
    cluster_var = (c_sum2 - n_c[:, None] * cluster_mu * cluster_mu) / denom
    cluster_std = jnp.sqrt(jnp.maximum(cluster_var, 0.0) + _EPS)

    cmu = jnp.dot(onehot, cluster_mu, preferred_element_type=jnp.float32)
    cstd = jnp.dot(onehot, cluster_std, preferred_element_type=jnp.float32)

    mu_mix = sample_mu * lm + cmu * (1.0 - lm)
    std_mix = sample_std * lm + cstd * (1.0 - lm)
    scale = std_mix / sample_std
    bias = mu_mix - sample_mu * scale
    scale_ref[:, :, 0] = scale
    bias_ref[:, :, 0] = bias


def _apply_body(x_ref, scale_ref, bias_ref, o_ref):
    o_ref[...] = x_ref[...] * scale_ref[...] + bias_ref[...]


def kernel(x, cluster_map):
    B, C, H, W = x.shape
    K = cluster_map.shape[2]
    HW = H * W
    xf = x.reshape(B, C, HW)

    lmda = jax.random.beta(jax.random.key(42), _ALPHA, _ALPHA, (B, 1, 1, 1)).astype(x.dtype)
    lm = lmda.reshape(B, 1, 1)

    NB = 4
    sums, sums2 = pl.pallas_call(
        _sums_body,
        grid=(B // NB,),
        in_specs=[pl.BlockSpec((NB, C, HW), lambda i: (i, 0, 0))],
        out_specs=[pl.BlockSpec((NB, C, 1), lambda i: (i, 0, 0)),
                   pl.BlockSpec((NB, C, 1), lambda i: (i, 0, 0))],
        out_shape=[jax.ShapeDtypeStruct((B, C, 1), jnp.float32),
                   jax.ShapeDtypeStruct((B, C, 1), jnp.float32)],
    )(xf)

    scale, bias = pl.pallas_call(
        _stats_body,
        out_shape=[jax.ShapeDtypeStruct((B, C, 1), jnp.float32),
                   jax.ShapeDtypeStruct((B, C, 1), jnp.float32)],
    )(cluster_map, lm, sums, sums2)

    out = pl.pallas_call(
        _apply_body,
        grid=(B // NB,),
        in_specs=[pl.BlockSpec((NB, C, HW), lambda i: (i, 0, 0)),
                  pl.BlockSpec((NB, C, 1), lambda i: (i, 0, 0)),
                  pl.BlockSpec((NB, C, 1), lambda i: (i, 0, 0))],
        out_specs=pl.BlockSpec((NB, C, HW), lambda i: (i, 0, 0)),
        out_shape=jax.ShapeDtypeStruct((B, C, HW), x.dtype),
    )(xf, scale, bias)

    return out.reshape(B, C, H, W)
